# baseline (speedup 1.0000x reference)
import functools

import jax
import jax.numpy as jnp
from jax import lax
from jax.experimental import pallas as pl
from jax.experimental.pallas import tpu as pltpu
from jax.experimental.pallas import tpu_sc as plsc

N_CLS = 1000
N_CTX = 16
D = 512
TOK = 76
TOK_PAD = 80
OUT_SEQ = 1 + N_CTX + TOK

NC = 2
NS = 16
NW = NC * NS

FULL_ROUNDS = N_CLS // NW
TAIL = N_CLS - FULL_ROUNDS * NW


def _body(idx_hbm, table_hbm, ctx0_hbm, prefix_hbm, out_hbm,
          idx_v, rows_v, head_v, sem):
  wid = lax.axis_index("s") * NC + lax.axis_index("c")

  pltpu.sync_copy(ctx0_hbm, head_v.at[pl.ds(1, N_CTX)])

  def do_class(i):
    pltpu.sync_copy(idx_hbm.at[i], idx_v)
    gather = pltpu.async_copy(table_hbm.at[idx_v], rows_v, sem)
    pltpu.sync_copy(prefix_hbm.at[i], head_v.at[0])
    pltpu.sync_copy(head_v, out_hbm.at[i].at[pl.ds(0, 1 + N_CTX)])
    gather.wait()
    pltpu.sync_copy(rows_v.at[pl.ds(0, TOK)],
                    out_hbm.at[i].at[pl.ds(1 + N_CTX, TOK)])

  def loop_body(t, carry):
    do_class(wid + t * NW)
    return carry

  lax.fori_loop(0, FULL_ROUNDS, loop_body, 0)

  @pl.when(wid < TAIL)
  def _():
    do_class(FULL_ROUNDS * NW + wid)


@jax.jit
def _run(idx, table, ctx0, prefix):
  mesh = plsc.VectorSubcoreMesh(
      core_axis_name="c", subcore_axis_name="s",
      num_cores=NC, num_subcores=NS)
  return pl.kernel(
      _body,
      out_type=jax.ShapeDtypeStruct((N_CLS, OUT_SEQ, D), jnp.float32),
      mesh=mesh,
      scratch_types=[
          pltpu.VMEM((TOK_PAD,), jnp.int32),
          pltpu.VMEM((TOK_PAD, D), jnp.float32),
          pltpu.VMEM((1 + N_CTX, D), jnp.float32),
          pltpu.SemaphoreType.DMA,
      ],
      compiler_params=pltpu.CompilerParams(use_tc_tiling_on_sc=False),
  )(idx, table, ctx0, prefix)


def kernel(tokenized_text, token_embedding, ctx, token_prefix):
  idx = tokenized_text[:, 1:].astype(jnp.int32)
  idx = jnp.concatenate(
      [idx, jnp.zeros((N_CLS, TOK_PAD - TOK), jnp.int32)], axis=1)
  ctx0 = ctx[0]
  prefix = token_prefix.reshape(N_CLS, D)
  return _run(idx, token_embedding, ctx0, prefix)
